# baseline (speedup 1.0000x reference)
import functools

import jax
import jax.numpy as jnp
from jax import lax
from jax.experimental import pallas as pl
from jax.experimental.pallas import tpu as pltpu
from jax.experimental.pallas import tpu_sc as plsc

N = 10000
G = 64
D = 128
H = 128
NPAD = 10240
NCORES = 1
NTILES = 16 * NCORES
CPT = NPAD // NTILES
NROWS = 12
WPT = NROWS * CPT
NCHUNKS = WPT // 128
AFLAT = (G + 1) * NPAD
ZFLAT = G * NPAD
ZPT = ZFLAT // 16
RPT = G // 16


def _sc_histogram(batch_pad, w_tiles):
    mesh = plsc.VectorSubcoreMesh(core_axis_name="c", subcore_axis_name="s",
                                  num_cores=NCORES)

    @functools.partial(
        pl.kernel,
        mesh=mesh,
        out_type=jax.ShapeDtypeStruct((NCORES, G + 1, NPAD), jnp.float32),
        scratch_types=[
            pltpu.VMEM((WPT,), jnp.int32),
            pltpu.VMEM((CPT,), jnp.int32),
            pltpu.VMEM((NCHUNKS, 128), jnp.int32),
            pltpu.VMEM((128,), jnp.float32),
            pltpu.VMEM((4096,), jnp.float32),
            pltpu.VMEM_SHARED((AFLAT,), jnp.float32),
            pltpu.VMEM_SHARED((NPAD,), jnp.int32),
            pltpu.SemaphoreType.DMA,
            pltpu.SemaphoreType.DMA,
        ],
    )
    def hist(batch_hbm, w_hbm, out_hbm,
             walk_v, pb_v, idx_v, ones_v, zbuf_v, a_sh, batch_sh,
             zsem, ssem):
        cid = lax.axis_index("c")
        sid = lax.axis_index("s")
        wid = cid * 16 + sid

        with jax.named_scope("sc_stage"):
            def zfill(i, _):
                zbuf_v[pl.ds(i * 16, 16)] = jnp.zeros((16,), jnp.float32)
                return _
            lax.fori_loop(0, 4096 // 16, zfill, 0)
            zcopies = [
                pltpu.async_copy(
                    zbuf_v,
                    a_sh.at[pl.ds(sid * ZPT + i * 4096, 4096)], zsem)
                for i in range(ZPT // 4096)]

            wcopies = [
                pltpu.async_copy(
                    w_hbm.at[pl.ds(k * NPAD + wid * CPT, CPT)],
                    walk_v.at[pl.ds(k * CPT, CPT)], ssem)
                for k in range(NROWS)]
            q = NPAD // 16
            bcopy = pltpu.async_copy(batch_hbm.at[pl.ds(sid * q, q)],
                                     batch_sh.at[pl.ds(sid * q, q)], ssem)

            for g in range(8):
                ones_v[pl.ds(g * 16, 16)] = jnp.ones((16,), jnp.float32)
            for w in wcopies:
                w.wait()
            bcopy.wait()

        with jax.named_scope("sc_zwait"):
            for z in zcopies:
                z.wait()
            plsc.subcore_barrier()

        with jax.named_scope("sc_pbgather"):
            gathers = [
                pltpu.async_copy(
                    batch_sh.at[walk_v.at[pl.ds(c * 128, ln)]],
                    pb_v.at[pl.ds(c * 128, ln)], ssem)
                for c, ln in [(o // 128, min(128, CPT - o))
                              for o in range(0, CPT, 128)]]
            for gth in gathers:
                gth.wait()

        with jax.named_scope("sc_scatter"):
            def idx_body(kc, _):
                for g in range(8):
                    p = kc * 128 + g * 16
                    j = lax.rem(p, CPT)
                    pb = pb_v[pl.ds(j, 16)]
                    wv = walk_v[pl.ds(p, 16)]
                    idx_v[kc, pl.ds(g * 16, 16)] = pb * NPAD + wv
                pltpu.async_copy(ones_v, a_sh.at[idx_v.at[kc]], ssem,
                                 add=True)
                return _
            lax.fori_loop(0, NCHUNKS, idx_body, 0)
            pltpu.make_async_copy(w_hbm.at[pl.ds(0, WPT)],
                                  walk_v, ssem).wait()

        with jax.named_scope("sc_bar2"):
            plsc.subcore_barrier()

        with jax.named_scope("sc_copyout"):
            for r in range(RPT):
                row = sid * RPT + r
                pltpu.sync_copy(a_sh.at[pl.ds(row * NPAD, NPAD)],
                                out_hbm.at[cid, row])

    return hist(batch_pad, w_tiles)


def _tc_body(a_ref, x_ref, w0_ref, b0_ref, w1_ref, b1_ref, w2_ref, b2_ref,
             o_ref, acc, srow):
    k = pl.program_id(0)

    @pl.when(k == 0)
    def _init():
        acc[...] = jnp.zeros_like(acc)
        srow[...] = jnp.zeros_like(srow)

    a = a_ref[0]
    for c in range(1, NCORES):
        a = a + a_ref[c]
    acc[...] += jnp.dot(a.astype(jnp.bfloat16), x_ref[...],
                        preferred_element_type=jnp.float32)
    srow[...] += jnp.sum(a, axis=1, keepdims=True)

    @pl.when(k == pl.num_programs(0) - 1)
    def _fin():
        b = acc[...]
        sv = srow[...]
        cdims = (((1,), (1,)), ((), ()))
        g0 = lax.dot_general(b, w0_ref[...], cdims,
                             preferred_element_type=jnp.float32)
        g0 = g0 + sv * b0_ref[...]
        g1 = lax.dot_general(g0, w1_ref[...], cdims,
                             preferred_element_type=jnp.float32)
        g1 = g1 + sv * b1_ref[...]
        g2 = lax.dot_general(g1, w2_ref[...], cdims,
                             preferred_element_type=jnp.float32)
        g2 = g2 + sv * b2_ref[...]
        o_ref[...] = jnp.concatenate([g0, g1, g2], axis=1)


def _tc_chain(a3, x_pad, W0, b0, W1, b1, W2, b2):
    kblk = 2560
    grid = (NPAD // kblk,)
    return pl.pallas_call(
        _tc_body,
        grid=grid,
        in_specs=[
            pl.BlockSpec((NCORES, G, kblk), lambda k: (0, 0, k)),
            pl.BlockSpec((kblk, D), lambda k: (k, 0)),
            pl.BlockSpec((H, D), lambda k: (0, 0)),
            pl.BlockSpec((1, H), lambda k: (0, 0)),
            pl.BlockSpec((H, H), lambda k: (0, 0)),
            pl.BlockSpec((1, H), lambda k: (0, 0)),
            pl.BlockSpec((H, H), lambda k: (0, 0)),
            pl.BlockSpec((1, H), lambda k: (0, 0)),
        ],
        out_specs=pl.BlockSpec((G, 3 * H), lambda k: (0, 0)),
        out_shape=jax.ShapeDtypeStruct((G, 3 * H), jnp.float32),
        scratch_shapes=[
            pltpu.VMEM((G, D), jnp.float32),
            pltpu.VMEM((G, 1), jnp.float32),
        ],
    )(a3, x_pad, W0, b0, W1, b1, W2, b2)


def kernel(x, edge_index, batch, walk2, walk3, walk4, W0, b0, W1, b1, W2, b2):
    del edge_index

    batch_pad = jnp.concatenate(
        [batch.astype(jnp.int32),
         jnp.full((NPAD - N,), G, jnp.int32)])
    w_all = jnp.concatenate(
        [walk2.astype(jnp.int32), walk3.astype(jnp.int32),
         walk4.astype(jnp.int32)], axis=0)
    pad_cols = NPAD - N
    w_pad = jnp.concatenate(
        [w_all,
         jnp.concatenate([jnp.full((1, pad_cols), N, jnp.int32),
                          jnp.zeros((NROWS - 1, pad_cols), jnp.int32)])],
        axis=1)
    a3 = _sc_histogram(batch_pad, w_pad.reshape(-1))

    x_pad = jnp.concatenate(
        [x.astype(jnp.bfloat16), jnp.zeros((NPAD - N, D), jnp.bfloat16)])
    return _tc_chain(a3, x_pad, W0, b0.reshape(1, H), W1, b1.reshape(1, H),
                     W2, b2.reshape(1, H))


# speedup vs baseline: 1.0259x; 1.0259x over previous
import functools

import jax
import jax.numpy as jnp
from jax import lax
from jax.experimental import pallas as pl
from jax.experimental.pallas import tpu as pltpu
from jax.experimental.pallas import tpu_sc as plsc

N = 10000
G = 64
D = 128
H = 128
NPAD = 10240
NCORES = 1
NTILES = 16 * NCORES
CPT = NPAD // NTILES
NROWS = 12
WPT = NROWS * CPT
NCHUNKS = WPT // 128
AFLAT = (G + 1) * NPAD
ZFLAT = G * NPAD
ZPT = ZFLAT // 16
RPT = G // 16


def _sc_histogram(batch_pad, w_tiles):
    mesh = plsc.VectorSubcoreMesh(core_axis_name="c", subcore_axis_name="s",
                                  num_cores=NCORES)

    @functools.partial(
        pl.kernel,
        mesh=mesh,
        out_type=jax.ShapeDtypeStruct((NCORES, G + 1, NPAD), jnp.float32),
        scratch_types=[
            pltpu.VMEM((WPT,), jnp.int32),
            pltpu.VMEM((CPT,), jnp.int32),
            pltpu.VMEM((NCHUNKS, 128), jnp.int32),
            pltpu.VMEM((128,), jnp.float32),
            pltpu.VMEM((4096,), jnp.float32),
            pltpu.VMEM_SHARED((AFLAT,), jnp.float32),
            pltpu.VMEM_SHARED((NPAD,), jnp.int32),
            pltpu.SemaphoreType.DMA,
            pltpu.SemaphoreType.DMA,
        ],
    )
    def hist(batch_hbm, w_hbm, out_hbm,
             walk_v, pb_v, idx_v, ones_v, zbuf_v, a_sh, batch_sh,
             zsem, ssem):
        cid = lax.axis_index("c")
        sid = lax.axis_index("s")
        wid = cid * 16 + sid

        with jax.named_scope("sc_stage"):
            def zfill(i, _):
                zbuf_v[pl.ds(i * 16, 16)] = jnp.zeros((16,), jnp.float32)
                return _
            lax.fori_loop(0, 4096 // 16, zfill, 0)
            zcopies = [
                pltpu.async_copy(
                    zbuf_v,
                    a_sh.at[pl.ds(sid * ZPT + i * 4096, 4096)], zsem)
                for i in range(ZPT // 4096)]

            wcopy = pltpu.async_copy(w_hbm.at[pl.ds(wid * WPT, WPT)],
                                     walk_v, ssem)
            q = NPAD // 16
            bcopy = pltpu.async_copy(batch_hbm.at[pl.ds(sid * q, q)],
                                     batch_sh.at[pl.ds(sid * q, q)], ssem)

            for g in range(8):
                ones_v[pl.ds(g * 16, 16)] = jnp.ones((16,), jnp.float32)
            wcopy.wait()
            bcopy.wait()

        with jax.named_scope("sc_zwait"):
            for z in zcopies:
                z.wait()
            plsc.subcore_barrier()

        with jax.named_scope("sc_pbgather"):
            gathers = [
                pltpu.async_copy(
                    batch_sh.at[walk_v.at[pl.ds(c * 128, ln)]],
                    pb_v.at[pl.ds(c * 128, ln)], ssem)
                for c, ln in [(o // 128, min(128, CPT - o))
                              for o in range(0, CPT, 128)]]
            for gth in gathers:
                gth.wait()

        with jax.named_scope("sc_scatter"):
            def idx_body(kc, _):
                for g in range(8):
                    p = kc * 128 + g * 16
                    j = lax.rem(p, CPT)
                    pb = pb_v[pl.ds(j, 16)]
                    wv = walk_v[pl.ds(p, 16)]
                    idx_v[kc, pl.ds(g * 16, 16)] = pb * NPAD + wv
                pltpu.async_copy(ones_v, a_sh.at[idx_v.at[kc]], ssem,
                                 add=True)
                return _
            lax.fori_loop(0, NCHUNKS, idx_body, 0)
            pltpu.make_async_copy(w_hbm.at[pl.ds(0, WPT)],
                                  walk_v, ssem).wait()

        with jax.named_scope("sc_bar2"):
            plsc.subcore_barrier()

        with jax.named_scope("sc_copyout"):
            for r in range(RPT):
                row = sid * RPT + r
                pltpu.sync_copy(a_sh.at[pl.ds(row * NPAD, NPAD)],
                                out_hbm.at[cid, row])

    return hist(batch_pad, w_tiles)


def _tc_body(a_ref, x_ref, w0_ref, b0_ref, w1_ref, b1_ref, w2_ref, b2_ref,
             o_ref, acc, srow):
    k = pl.program_id(0)

    @pl.when(k == 0)
    def _init():
        acc[...] = jnp.zeros_like(acc)
        srow[...] = jnp.zeros_like(srow)

    a = a_ref[0]
    for c in range(1, NCORES):
        a = a + a_ref[c]
    acc[...] += jnp.dot(a.astype(jnp.bfloat16), x_ref[...],
                        preferred_element_type=jnp.float32)
    srow[...] += jnp.sum(a, axis=1, keepdims=True)

    @pl.when(k == pl.num_programs(0) - 1)
    def _fin():
        b = acc[...]
        sv = srow[...]
        cdims = (((1,), (1,)), ((), ()))
        g0 = lax.dot_general(b, w0_ref[...], cdims,
                             preferred_element_type=jnp.float32)
        g0 = g0 + sv * b0_ref[...]
        g1 = lax.dot_general(g0, w1_ref[...], cdims,
                             preferred_element_type=jnp.float32)
        g1 = g1 + sv * b1_ref[...]
        g2 = lax.dot_general(g1, w2_ref[...], cdims,
                             preferred_element_type=jnp.float32)
        g2 = g2 + sv * b2_ref[...]
        o_ref[...] = jnp.concatenate([g0, g1, g2], axis=1)


def _tc_chain(a3, x_pad, W0, b0, W1, b1, W2, b2):
    kblk = 2560
    grid = (NPAD // kblk,)
    return pl.pallas_call(
        _tc_body,
        grid=grid,
        in_specs=[
            pl.BlockSpec((NCORES, G, kblk), lambda k: (0, 0, k)),
            pl.BlockSpec((kblk, D), lambda k: (k, 0)),
            pl.BlockSpec((H, D), lambda k: (0, 0)),
            pl.BlockSpec((1, H), lambda k: (0, 0)),
            pl.BlockSpec((H, H), lambda k: (0, 0)),
            pl.BlockSpec((1, H), lambda k: (0, 0)),
            pl.BlockSpec((H, H), lambda k: (0, 0)),
            pl.BlockSpec((1, H), lambda k: (0, 0)),
        ],
        out_specs=pl.BlockSpec((G, 3 * H), lambda k: (0, 0)),
        out_shape=jax.ShapeDtypeStruct((G, 3 * H), jnp.float32),
        scratch_shapes=[
            pltpu.VMEM((G, D), jnp.float32),
            pltpu.VMEM((G, 1), jnp.float32),
        ],
    )(a3, x_pad, W0, b0, W1, b1, W2, b2)


def kernel(x, edge_index, batch, walk2, walk3, walk4, W0, b0, W1, b1, W2, b2):
    del edge_index

    batch_pad = jnp.concatenate(
        [batch.astype(jnp.int32),
         jnp.full((NPAD - N,), G, jnp.int32)])
    w_all = jnp.concatenate(
        [walk2.astype(jnp.int32), walk3.astype(jnp.int32),
         walk4.astype(jnp.int32)], axis=0)
    pad_cols = NPAD - N
    w_pad = jnp.concatenate(
        [w_all,
         jnp.concatenate([jnp.full((1, pad_cols), N, jnp.int32),
                          jnp.zeros((NROWS - 1, pad_cols), jnp.int32)])],
        axis=1)
    w_tiles = (w_pad.reshape(NROWS, NTILES, CPT)
               .transpose(1, 0, 2).reshape(-1))

    a3 = _sc_histogram(batch_pad, w_tiles)

    x_pad = jnp.concatenate(
        [x.astype(jnp.bfloat16), jnp.zeros((NPAD - N, D), jnp.bfloat16)])
    return _tc_chain(a3, x_pad, W0, b0.reshape(1, H), W1, b1.reshape(1, H),
                     W2, b2.reshape(1, H))
